# Initial kernel scaffold; baseline (speedup 1.0000x reference)
import jax
import jax.numpy as jnp
from jax.experimental import pallas as pl


def kernel(point_token, patch_center, image_patch_token, image_patch_coord, Wi1, bi1, Wi2, bi2, Wg1, bg1, Wg2, bg2, Wd1, bd1, Wd2, bd2):
    raise NotImplementedError("write your pallas kernel here")


# speedup vs baseline: 21.1038x; 21.1038x over previous
import functools

import jax
import jax.numpy as jnp
from jax.experimental import pallas as pl
from jax.experimental.pallas import tpu as pltpu

K = 3
EPS = 1e-06


def _body(pt_ref, pc_ref, it_ref, ic_ref,
          wi1_ref, bi1_ref, wi2_ref, bi2_ref,
          wg1_ref, bg1_ref, wg2_ref, bg2_ref,
          wd1_ref, bd1_ref, wd2_ref, bd2_ref,
          out_ref, feat_ref, *, n_img):
    j = pl.program_id(1)

    @pl.when(j == 0)
    def _():
        x = it_ref[0]
        h = jnp.dot(x, wi1_ref[...], preferred_element_type=jnp.float32)
        h = jnp.maximum(h + bi1_ref[...], 0.0)
        feat_ref[...] = (jnp.dot(h, wi2_ref[...], preferred_element_type=jnp.float32)
                         + bi2_ref[...])

    q = pc_ref[0]
    s = ic_ref[0]
    q_sq = jnp.sum(q * q, axis=1, keepdims=True)
    s_sq = jnp.sum(s * s, axis=1, keepdims=True)
    cross = jax.lax.dot_general(q, s, (((1,), (1,)), ((), ())),
                                preferred_element_type=jnp.float32)
    sqd = jnp.maximum(q_sq + s_sq.T - 2.0 * cross, 0.0)

    iota = jax.lax.broadcasted_iota(jnp.int32, sqd.shape, 1)
    d = sqd
    ws = []
    onehots = []
    for _ in range(K):
        m = jnp.min(d, axis=1, keepdims=True)
        idx = jnp.min(jnp.where(d == m, iota, n_img), axis=1, keepdims=True)
        sel = iota == idx
        d = jnp.where(sel, jnp.float32(3.0e38), d)
        dist = jnp.sqrt(m)
        ws.append(1.0 / jnp.maximum(dist, EPS))
        onehots.append(sel)
    wsum = jnp.maximum(ws[0] + ws[1] + ws[2], EPS)
    comb = jnp.zeros_like(sqd)
    for w, sel in zip(ws, onehots):
        comb = jnp.where(sel, w / wsum, comb)

    aligned = jnp.dot(comb, feat_ref[...], preferred_element_type=jnp.float32)

    point = pt_ref[0]
    x = jnp.concatenate([point, aligned], axis=1)

    hg = jnp.maximum(jnp.dot(x, wg1_ref[...], preferred_element_type=jnp.float32)
                     + bg1_ref[...], 0.0)
    gate = jax.nn.sigmoid(jnp.dot(hg, wg2_ref[...], preferred_element_type=jnp.float32)
                          + bg2_ref[...])
    hd_ = jnp.maximum(jnp.dot(x, wd1_ref[...], preferred_element_type=jnp.float32)
                      + bd1_ref[...], 0.0)
    delta = (jnp.dot(hd_, wd2_ref[...], preferred_element_type=jnp.float32)
             + bd2_ref[...])

    out_ref[0] = point + gate * delta


def kernel(point_token, patch_center, image_patch_token, image_patch_coord,
           Wi1, bi1, Wi2, bi2, Wg1, bg1, Wg2, bg2, Wd1, bd1, Wd2, bd2):
    B, Np, od = point_token.shape
    Ni, idim = image_patch_token.shape[1:]
    hd = Wi1.shape[1]
    BN = min(512, Np)

    b2 = lambda b: b.reshape(1, -1)

    full = lambda arr: pl.BlockSpec(arr.shape, lambda b, j: (0,) * arr.ndim)
    grid = (B, Np // BN)

    out = pl.pallas_call(
        functools.partial(_body, n_img=Ni),
        grid=grid,
        in_specs=[
            pl.BlockSpec((1, BN, od), lambda b, j: (b, j, 0)),
            pl.BlockSpec((1, BN, 3), lambda b, j: (b, j, 0)),
            pl.BlockSpec((1, Ni, idim), lambda b, j: (b, 0, 0)),
            pl.BlockSpec((1, Ni, 3), lambda b, j: (b, 0, 0)),
            full(Wi1), pl.BlockSpec((1, hd), lambda b, j: (0, 0)),
            full(Wi2), pl.BlockSpec((1, od), lambda b, j: (0, 0)),
            full(Wg1), pl.BlockSpec((1, hd), lambda b, j: (0, 0)),
            full(Wg2), pl.BlockSpec((1, od), lambda b, j: (0, 0)),
            full(Wd1), pl.BlockSpec((1, hd), lambda b, j: (0, 0)),
            full(Wd2), pl.BlockSpec((1, od), lambda b, j: (0, 0)),
        ],
        out_specs=pl.BlockSpec((1, BN, od), lambda b, j: (b, j, 0)),
        out_shape=jax.ShapeDtypeStruct((B, Np, od), jnp.float32),
        scratch_shapes=[pltpu.VMEM((Ni, od), jnp.float32)],
        compiler_params=pltpu.CompilerParams(
            dimension_semantics=("arbitrary", "arbitrary")),
    )(point_token, patch_center, image_patch_token, image_patch_coord,
      Wi1, b2(bi1), Wi2, b2(bi2), Wg1, b2(bg1), Wg2, b2(bg2),
      Wd1, b2(bd1), Wd2, b2(bd2))
    return out


# speedup vs baseline: 22.3604x; 1.0595x over previous
import functools

import jax
import jax.numpy as jnp
from jax.experimental import pallas as pl
from jax.experimental.pallas import tpu as pltpu

K = 3
EPS = 1e-06


def _body(pt_ref, pc_ref, it_ref, ic_ref,
          wi1_ref, bi1_ref, wi2_ref, bi2_ref,
          wg1_ref, bg1_ref, wg2_ref, bg2_ref,
          wd1_ref, bd1_ref, wd2_ref, bd2_ref,
          out_ref, feat_ref, *, n_img):
    j = pl.program_id(1)

    @pl.when(j == 0)
    def _():
        x = it_ref[0]
        h = jnp.dot(x, wi1_ref[...], preferred_element_type=jnp.float32)
        h = jnp.maximum(h + bi1_ref[...], 0.0)
        feat_ref[...] = (jnp.dot(h, wi2_ref[...], preferred_element_type=jnp.float32)
                         + bi2_ref[...])

    q = pc_ref[0]
    s = ic_ref[0]
    q_sq = jnp.sum(q * q, axis=1, keepdims=True)
    s_sq = jnp.sum(s * s, axis=1, keepdims=True)
    cross = jax.lax.dot_general(q, s, (((1,), (1,)), ((), ())),
                                preferred_element_type=jnp.float32)
    sqd = jnp.maximum(q_sq + s_sq.T - 2.0 * cross, 0.0)

    iota = jax.lax.broadcasted_iota(jnp.int32, sqd.shape, 1)
    d = sqd
    ws = []
    onehots = []
    for _ in range(K):
        m = jnp.min(d, axis=1, keepdims=True)
        idx = jnp.min(jnp.where(d == m, iota, n_img), axis=1, keepdims=True)
        sel = iota == idx
        d = jnp.where(sel, jnp.float32(3.0e38), d)
        dist = jnp.sqrt(m)
        ws.append(1.0 / jnp.maximum(dist, EPS))
        onehots.append(sel)
    wsum = jnp.maximum(ws[0] + ws[1] + ws[2], EPS)
    comb = jnp.zeros_like(sqd)
    for w, sel in zip(ws, onehots):
        comb = jnp.where(sel, w / wsum, comb)

    aligned = jnp.dot(comb, feat_ref[...], preferred_element_type=jnp.float32)

    point = pt_ref[0]
    x = jnp.concatenate([point, aligned], axis=1)

    hg = jnp.maximum(jnp.dot(x, wg1_ref[...], preferred_element_type=jnp.float32)
                     + bg1_ref[...], 0.0)
    gate = jax.nn.sigmoid(jnp.dot(hg, wg2_ref[...], preferred_element_type=jnp.float32)
                          + bg2_ref[...])
    hd_ = jnp.maximum(jnp.dot(x, wd1_ref[...], preferred_element_type=jnp.float32)
                      + bd1_ref[...], 0.0)
    delta = (jnp.dot(hd_, wd2_ref[...], preferred_element_type=jnp.float32)
             + bd2_ref[...])

    out_ref[0] = point + gate * delta


def kernel(point_token, patch_center, image_patch_token, image_patch_coord,
           Wi1, bi1, Wi2, bi2, Wg1, bg1, Wg2, bg2, Wd1, bd1, Wd2, bd2):
    B, Np, od = point_token.shape
    Ni, idim = image_patch_token.shape[1:]
    hd = Wi1.shape[1]
    BN = min(1024, Np)

    b2 = lambda b: b.reshape(1, -1)

    full = lambda arr: pl.BlockSpec(arr.shape, lambda b, j: (0,) * arr.ndim)
    grid = (B, Np // BN)

    out = pl.pallas_call(
        functools.partial(_body, n_img=Ni),
        grid=grid,
        in_specs=[
            pl.BlockSpec((1, BN, od), lambda b, j: (b, j, 0)),
            pl.BlockSpec((1, BN, 3), lambda b, j: (b, j, 0)),
            pl.BlockSpec((1, Ni, idim), lambda b, j: (b, 0, 0)),
            pl.BlockSpec((1, Ni, 3), lambda b, j: (b, 0, 0)),
            full(Wi1), pl.BlockSpec((1, hd), lambda b, j: (0, 0)),
            full(Wi2), pl.BlockSpec((1, od), lambda b, j: (0, 0)),
            full(Wg1), pl.BlockSpec((1, hd), lambda b, j: (0, 0)),
            full(Wg2), pl.BlockSpec((1, od), lambda b, j: (0, 0)),
            full(Wd1), pl.BlockSpec((1, hd), lambda b, j: (0, 0)),
            full(Wd2), pl.BlockSpec((1, od), lambda b, j: (0, 0)),
        ],
        out_specs=pl.BlockSpec((1, BN, od), lambda b, j: (b, j, 0)),
        out_shape=jax.ShapeDtypeStruct((B, Np, od), jnp.float32),
        scratch_shapes=[pltpu.VMEM((Ni, od), jnp.float32)],
        compiler_params=pltpu.CompilerParams(
            dimension_semantics=("arbitrary", "arbitrary")),
    )(point_token, patch_center, image_patch_token, image_patch_coord,
      Wi1, b2(bi1), Wi2, b2(bi2), Wg1, b2(bg1), Wg2, b2(bg2),
      Wd1, b2(bd1), Wd2, b2(bd2))
    return out
